# baseline (speedup 1.0000x reference)
import functools
import math

import jax
import jax.numpy as jnp
import numpy as np
from jax import lax
from jax.experimental import pallas as pl
from jax.experimental.pallas import tpu as pltpu
from jax.experimental.pallas import tpu_sc as plsc

VOCAB = 1000000
DIM = 64
PAD = 128
SEQ = 200
BATCH = 4096

NC = 2
NS = 16
NW = NC * NS
LANES = 16
VPER = DIM // LANES

CHUNK_A = 200
N_CHUNKS_A = 158

B_PER_W = BATCH // NW
SUB_OFFS = (0, 104)
SUB_LENS = (104, 96)


def _pos_encoding():
    pos = np.arange(SEQ, dtype=np.float32)[:, None]
    fill = pos * np.exp(
        -np.arange(0, DIM, 2, dtype=np.float32) * math.log(10000.0) / DIM
    )
    enc = np.zeros((SEQ, DIM), dtype=np.float32)
    enc[:, 0::2] = np.sin(fill)
    enc[:, 1::2] = np.cos(fill)
    return enc


_mesh = plsc.VectorSubcoreMesh(core_axis_name="c", subcore_axis_name="s")
_params = pltpu.CompilerParams(use_tc_tiling_on_sc=True)


@functools.partial(
    pl.kernel,
    out_type=jax.ShapeDtypeStruct((VOCAB, PAD), jnp.float32),
    mesh=_mesh,
    scratch_types=[
        [pltpu.VMEM((CHUNK_A, DIM), jnp.float32) for _ in range(2)],
        [pltpu.VMEM((CHUNK_A, PAD), jnp.float32) for _ in range(2)],
        pltpu.SemaphoreType.DMA((2,)),
        pltpu.SemaphoreType.DMA((2,)),
    ],
    compiler_params=_params,
)
def _repack_kernel(src_hbm, dst_hbm, rbufs, wbufs, rsem, wsem):
    wid = lax.axis_index("s") * NC + lax.axis_index("c")
    start = wid * 156 + jnp.minimum(wid, 8)

    def _off(c):
        raw = jnp.minimum((start + c) * CHUNK_A, VOCAB - CHUNK_A)
        return pl.multiple_of(raw, 8)

    def fire_read(c, b):
        pltpu.async_copy(
            src_hbm.at[pl.ds(_off(c), CHUNK_A)], rbufs[b], rsem.at[b]
        )

    def wait_read(b):
        pltpu.make_async_copy(
            src_hbm.at[pl.ds(0, CHUNK_A)], rbufs[b], rsem.at[b]
        ).wait()

    def fire_write(c, b):
        pltpu.async_copy(
            wbufs[b], dst_hbm.at[pl.ds(_off(c), CHUNK_A)], wsem.at[b]
        )

    def wait_write(b):
        pltpu.make_async_copy(
            wbufs[b], dst_hbm.at[pl.ds(0, CHUNK_A)], wsem.at[b]
        ).wait()

    fire_read(0, 0)

    def step(p, carry):
        for b in range(2):
            c = p * 2 + b
            nb = 1 - b
            pl.when(p > 0)(lambda b=b: wait_write(b))
            if b == 0:
                fire_read(c + 1, nb)
            else:
                pl.when(p < (N_CHUNKS_A // 2) - 1)(
                    lambda c=c, nb=nb: fire_read(c + 1, nb)
                )
            wait_read(b)
            rb, wb = rbufs[b], wbufs[b]

            @plsc.parallel_loop(0, CHUNK_A, unroll=4)
            def _(r):
                for j in range(VPER):
                    sl = pl.ds(j * LANES, LANES)
                    wb[r, sl] = rb[r, sl]

            fire_write(c, b)
        return carry

    lax.fori_loop(0, N_CHUNKS_A // 2, step, 0)
    wait_write(0)
    wait_write(1)


@functools.partial(
    pl.kernel,
    out_type=jax.ShapeDtypeStruct((BATCH, SEQ, DIM), jnp.float32),
    mesh=_mesh,
    scratch_types=[
        pltpu.VMEM((SEQ, DIM), jnp.float32),
        [pltpu.VMEM((SEQ,), jnp.int32) for _ in range(4)],
        [pltpu.VMEM((SEQ, PAD), jnp.float32) for _ in range(2)],
        [pltpu.VMEM((SEQ, DIM), jnp.float32) for _ in range(2)],
        pltpu.SemaphoreType.DMA((4,)),
        pltpu.SemaphoreType.DMA((2,)),
        pltpu.SemaphoreType.DMA((2,)),
    ],
    compiler_params=_params,
)
def _gather_kernel(
    x_hbm, table_hbm, pos_hbm, out_hbm, pos_v, ibufs, rows, outs, isem, gsem, ssem
):
    wid = lax.axis_index("s") * NC + lax.axis_index("c")
    bbase = wid * B_PER_W
    pltpu.sync_copy(pos_hbm, pos_v)

    def fire_idx(c, q):
        off = pl.multiple_of((bbase + c) * SEQ, 8)
        pltpu.async_copy(x_hbm.at[pl.ds(off, SEQ)], ibufs[q], isem.at[q])

    def wait_idx(q):
        pltpu.make_async_copy(
            x_hbm.at[pl.ds(0, SEQ)], ibufs[q], isem.at[q]
        ).wait()

    def fire_gather(c, q, b):
        for o, n in zip(SUB_OFFS, SUB_LENS):
            pltpu.async_copy(
                table_hbm.at[ibufs[q].at[pl.ds(o, n)]],
                rows[b].at[pl.ds(o, n)],
                gsem.at[b],
            )

    def wait_gather(b):
        pltpu.make_async_copy(
            table_hbm.at[pl.ds(0, SEQ)], rows[b], gsem.at[b]
        ).wait()

    def fire_store(c, b):
        pltpu.async_copy(outs[b], out_hbm.at[bbase + c], ssem.at[b])

    def wait_store(b):
        pltpu.make_async_copy(outs[b], out_hbm.at[0], ssem.at[b]).wait()

    fire_idx(0, 0)
    wait_idx(0)
    fire_gather(0, 0, 0)
    fire_idx(1, 1)

    NP = B_PER_W // 4

    def step(p, carry):
        for b in range(4):
            c = p * 4 + b
            mb = b % 2
            if b < 2:
                pl.when(p > 0)(lambda mb=mb: wait_store(mb))
            else:
                wait_store(mb)
            def _pref(c=c, q=(b + 2) % 4):
                fire_idx(c + 2, q)

            if b < 2:
                _pref()
            else:
                pl.when(p < NP - 1)(_pref)
            def _gnext(c=c, q=(b + 1) % 4, nb=(b + 1) % 2):
                wait_idx(q)
                fire_gather(c + 1, q, nb)

            if b < 3:
                _gnext()
            else:
                pl.when(p < NP - 1)(_gnext)
            wait_gather(mb)
            g, o = rows[mb], outs[mb]

            @plsc.parallel_loop(0, SEQ, unroll=4)
            def _(r):
                for j in range(VPER):
                    sl = pl.ds(j * LANES, LANES)
                    o[r, sl] = g[r, sl] + pos_v[r, sl]

            fire_store(c, b % 2)
        return carry

    lax.fori_loop(0, NP, step, 0)
    wait_store(0)
    wait_store(1)


def kernel(x, emb_table):
    pos = jnp.asarray(_pos_encoding())
    table_pad = jnp.concatenate([emb_table, emb_table], axis=1)
    x_flat = x.reshape(BATCH * SEQ).astype(jnp.int32)
    return _gather_kernel(x_flat, table_pad, pos)


# speedup vs baseline: 1.0004x; 1.0004x over previous
import functools
import math

import jax
import jax.numpy as jnp
import numpy as np
from jax import lax
from jax.experimental import pallas as pl
from jax.experimental.pallas import tpu as pltpu
from jax.experimental.pallas import tpu_sc as plsc

VOCAB = 1000000
DIM = 64
PAD = 128
SEQ = 200
BATCH = 4096

NC = 2
NS = 16
NW = NC * NS
LANES = 16
VPER = DIM // LANES

CHUNK_A = 200
N_CHUNKS_A = 158

B_PER_W = BATCH // NW
SUB_OFFS = (0, 104)
SUB_LENS = (104, 96)


def _pos_encoding():
    pos = np.arange(SEQ, dtype=np.float32)[:, None]
    fill = pos * np.exp(
        -np.arange(0, DIM, 2, dtype=np.float32) * math.log(10000.0) / DIM
    )
    enc = np.zeros((SEQ, DIM), dtype=np.float32)
    enc[:, 0::2] = np.sin(fill)
    enc[:, 1::2] = np.cos(fill)
    return enc


_mesh = plsc.VectorSubcoreMesh(core_axis_name="c", subcore_axis_name="s")
_params = pltpu.CompilerParams(use_tc_tiling_on_sc=True)


@functools.partial(
    pl.kernel,
    out_type=jax.ShapeDtypeStruct((VOCAB, PAD), jnp.float32),
    mesh=_mesh,
    scratch_types=[
        [pltpu.VMEM((CHUNK_A, DIM), jnp.float32) for _ in range(2)],
        [pltpu.VMEM((CHUNK_A, PAD), jnp.float32) for _ in range(2)],
        pltpu.SemaphoreType.DMA((2,)),
        pltpu.SemaphoreType.DMA((2,)),
    ],
    compiler_params=_params,
)
def _repack_kernel(src_hbm, dst_hbm, rbufs, wbufs, rsem, wsem):
    wid = lax.axis_index("s") * NC + lax.axis_index("c")
    start = wid * 156 + jnp.minimum(wid, 8)

    def _off(c):
        raw = jnp.minimum((start + c) * CHUNK_A, VOCAB - CHUNK_A)
        return pl.multiple_of(raw, 8)

    def fire_read(c, b):
        pltpu.async_copy(
            src_hbm.at[pl.ds(_off(c), CHUNK_A)], rbufs[b], rsem.at[b]
        )

    def wait_read(b):
        pltpu.make_async_copy(
            src_hbm.at[pl.ds(0, CHUNK_A)], rbufs[b], rsem.at[b]
        ).wait()

    def fire_write(c, b):
        pltpu.async_copy(
            wbufs[b], dst_hbm.at[pl.ds(_off(c), CHUNK_A)], wsem.at[b]
        )

    def wait_write(b):
        pltpu.make_async_copy(
            wbufs[b], dst_hbm.at[pl.ds(0, CHUNK_A)], wsem.at[b]
        ).wait()

    fire_read(0, 0)

    def step(p, carry):
        for b in range(2):
            c = p * 2 + b
            nb = 1 - b
            pl.when(p > 0)(lambda b=b: wait_write(b))
            if b == 0:
                fire_read(c + 1, nb)
            else:
                pl.when(p < (N_CHUNKS_A // 2) - 1)(
                    lambda c=c, nb=nb: fire_read(c + 1, nb)
                )
            wait_read(b)
            rb, wb = rbufs[b], wbufs[b]

            @plsc.parallel_loop(0, CHUNK_A, unroll=4)
            def _(r):
                for j in range(VPER):
                    sl = pl.ds(j * LANES, LANES)
                    wb[r, sl] = rb[r, sl]

            fire_write(c, b)
        return carry

    lax.fori_loop(0, N_CHUNKS_A // 2, step, 0)
    wait_write(0)
    wait_write(1)


@functools.partial(
    pl.kernel,
    out_type=jax.ShapeDtypeStruct((BATCH, SEQ, PAD), jnp.float32),
    mesh=_mesh,
    scratch_types=[
        [pltpu.VMEM((SEQ,), jnp.int32) for _ in range(4)],
        [pltpu.VMEM((SEQ, PAD), jnp.float32) for _ in range(4)],
        pltpu.SemaphoreType.DMA((4,)),
        pltpu.SemaphoreType.DMA((4,)),
        pltpu.SemaphoreType.DMA((4,)),
    ],
    compiler_params=_params,
)
def _gather_kernel(
    x_hbm, table_hbm, out_hbm, ibufs, rows, isem, gsem, ssem
):
    wid = lax.axis_index("s") * NC + lax.axis_index("c")
    bbase = wid * B_PER_W

    def fire_idx(c, q):
        off = pl.multiple_of((bbase + c) * SEQ, 8)
        pltpu.async_copy(x_hbm.at[pl.ds(off, SEQ)], ibufs[q], isem.at[q])

    def wait_idx(q):
        pltpu.make_async_copy(
            x_hbm.at[pl.ds(0, SEQ)], ibufs[q], isem.at[q]
        ).wait()

    def fire_gather(c, q, b):
        for o, n in zip(SUB_OFFS, SUB_LENS):
            pltpu.async_copy(
                table_hbm.at[ibufs[q].at[pl.ds(o, n)]],
                rows[b].at[pl.ds(o, n)],
                gsem.at[b],
            )

    def wait_gather(b):
        pltpu.make_async_copy(
            table_hbm.at[pl.ds(0, SEQ)], rows[b], gsem.at[b]
        ).wait()

    def fire_store(c, b):
        pltpu.async_copy(rows[b], out_hbm.at[bbase + c], ssem.at[b])

    def wait_store(b):
        pltpu.make_async_copy(rows[b], out_hbm.at[0], ssem.at[b]).wait()

    fire_idx(0, 0)
    wait_idx(0)
    fire_gather(0, 0, 0)
    fire_idx(1, 1)

    NP = B_PER_W // 4

    def step(p, carry):
        for b in range(4):
            c = p * 4 + b
            def _pref(c=c, q=(b + 2) % 4):
                fire_idx(c + 2, q)

            if b < 2:
                _pref()
            else:
                pl.when(p < NP - 1)(_pref)

            def _gnext(c=c, q=(b + 1) % 4, nb=(b + 1) % 4):
                fire_gather(c + 1, q, nb)

            def _wstore(nb=(b + 1) % 4):
                wait_store(nb)

            def _widx(q=(b + 1) % 4):
                wait_idx(q)

            if b < 3:
                pl.when(p > 0)(_wstore)
                _widx()
                _gnext()
            else:
                def _tail():
                    _wstore()
                    _widx()
                    _gnext()

                pl.when(p < NP - 1)(_tail)

            wait_gather(b)
            fire_store(c, b)
        return carry

    lax.fori_loop(0, NP, step, 0)
    for b in range(4):
        wait_store(b)


def kernel(x, emb_table):
    pos = jnp.asarray(_pos_encoding())
    table_pad = jnp.pad(emb_table, ((0, 0), (0, PAD - DIM)))
    x_flat = x.reshape(BATCH * SEQ).astype(jnp.int32)
    raw = _gather_kernel(x_flat, table_pad)
    return raw[:, :, :DIM] + pos[None, :, :]


# speedup vs baseline: 1.1351x; 1.1347x over previous
import functools
import math

import jax
import jax.numpy as jnp
import numpy as np
from jax import lax
from jax.experimental import pallas as pl
from jax.experimental.pallas import tpu as pltpu
from jax.experimental.pallas import tpu_sc as plsc

VOCAB = 1000000
DIM = 64
PAD = 128
SEQ = 200
BATCH = 4096

NC = 2
NS = 16
NW = NC * NS
LANES = 16
VPER = DIM // LANES

CHUNK_A = 200
N_CHUNKS_A = 158

B_PER_W = BATCH // NW
SUB_OFFS = (0, 104)
SUB_LENS = (104, 96)


def _pos_encoding():
    pos = np.arange(SEQ, dtype=np.float32)[:, None]
    fill = pos * np.exp(
        -np.arange(0, DIM, 2, dtype=np.float32) * math.log(10000.0) / DIM
    )
    enc = np.zeros((SEQ, DIM), dtype=np.float32)
    enc[:, 0::2] = np.sin(fill)
    enc[:, 1::2] = np.cos(fill)
    return enc


_mesh = plsc.VectorSubcoreMesh(core_axis_name="c", subcore_axis_name="s")
_params = pltpu.CompilerParams(use_tc_tiling_on_sc=True)


@functools.partial(
    pl.kernel,
    out_type=jax.ShapeDtypeStruct((VOCAB, PAD), jnp.float32),
    mesh=_mesh,
    scratch_types=[
        [pltpu.VMEM((CHUNK_A, DIM), jnp.float32) for _ in range(2)],
        [pltpu.VMEM((CHUNK_A, PAD), jnp.float32) for _ in range(2)],
        pltpu.SemaphoreType.DMA((2,)),
        pltpu.SemaphoreType.DMA((2,)),
    ],
    compiler_params=_params,
)
def _repack_kernel(src_hbm, dst_hbm, rbufs, wbufs, rsem, wsem):
    wid = lax.axis_index("s") * NC + lax.axis_index("c")
    start = wid * 156 + jnp.minimum(wid, 8)

    def _off(c):
        raw = jnp.minimum((start + c) * CHUNK_A, VOCAB - CHUNK_A)
        return pl.multiple_of(raw, 8)

    def fire_read(c, b):
        pltpu.async_copy(
            src_hbm.at[pl.ds(_off(c), CHUNK_A)], rbufs[b], rsem.at[b]
        )

    def wait_read(b):
        pltpu.make_async_copy(
            src_hbm.at[pl.ds(0, CHUNK_A)], rbufs[b], rsem.at[b]
        ).wait()

    def fire_write(c, b):
        pltpu.async_copy(
            wbufs[b], dst_hbm.at[pl.ds(_off(c), CHUNK_A)], wsem.at[b]
        )

    def wait_write(b):
        pltpu.make_async_copy(
            wbufs[b], dst_hbm.at[pl.ds(0, CHUNK_A)], wsem.at[b]
        ).wait()

    fire_read(0, 0)

    def step(p, carry):
        for b in range(2):
            c = p * 2 + b
            nb = 1 - b
            pl.when(p > 0)(lambda b=b: wait_write(b))
            if b == 0:
                fire_read(c + 1, nb)
            else:
                pl.when(p < (N_CHUNKS_A // 2) - 1)(
                    lambda c=c, nb=nb: fire_read(c + 1, nb)
                )
            wait_read(b)
            rb, wb = rbufs[b], wbufs[b]

            @plsc.parallel_loop(0, CHUNK_A, unroll=4)
            def _(r):
                for j in range(VPER):
                    sl = pl.ds(j * LANES, LANES)
                    wb[r, sl] = rb[r, sl]

            fire_write(c, b)
        return carry

    lax.fori_loop(0, N_CHUNKS_A // 2, step, 0)
    wait_write(0)
    wait_write(1)


@functools.partial(
    pl.kernel,
    out_type=jax.ShapeDtypeStruct((BATCH, SEQ, DIM), jnp.float32),
    mesh=_mesh,
    scratch_types=[
        pltpu.VMEM((SEQ, DIM), jnp.float32),
        [pltpu.VMEM((SEQ,), jnp.int32) for _ in range(4)],
        [pltpu.VMEM((SEQ, PAD), jnp.float32) for _ in range(2)],
        [pltpu.VMEM((SEQ, DIM), jnp.float32) for _ in range(2)],
        pltpu.SemaphoreType.DMA((4,)),
        pltpu.SemaphoreType.DMA((2,)),
        pltpu.SemaphoreType.DMA((2,)),
    ],
    compiler_params=_params,
)
def _gather_kernel(
    x_hbm, table_hbm, pos_hbm, out_hbm, pos_v, ibufs, rows, outs, isem, gsem, ssem
):
    wid = lax.axis_index("s") * NC + lax.axis_index("c")
    bbase = wid * B_PER_W
    pltpu.sync_copy(pos_hbm, pos_v)

    def fire_idx(c, q):
        off = pl.multiple_of((bbase + c) * SEQ, 8)
        pltpu.async_copy(x_hbm.at[pl.ds(off, SEQ)], ibufs[q], isem.at[q])

    def wait_idx(q):
        pltpu.make_async_copy(
            x_hbm.at[pl.ds(0, SEQ)], ibufs[q], isem.at[q]
        ).wait()

    def fire_gather(c, q, b):
        for o, n in zip(SUB_OFFS, SUB_LENS):
            pltpu.async_copy(
                table_hbm.at[ibufs[q].at[pl.ds(o, n)]],
                rows[b].at[pl.ds(o, n)],
                gsem.at[b],
            )

    def wait_gather(b):
        pltpu.make_async_copy(
            table_hbm.at[pl.ds(0, SEQ)], rows[b], gsem.at[b]
        ).wait()

    def fire_store(c, b):
        pltpu.async_copy(outs[b], out_hbm.at[bbase + c], ssem.at[b])

    def wait_store(b):
        pltpu.make_async_copy(outs[b], out_hbm.at[0], ssem.at[b]).wait()

    fire_idx(0, 0)
    wait_idx(0)
    fire_gather(0, 0, 0)
    fire_idx(1, 1)

    NP = B_PER_W // 4

    def step(p, carry):
        for b in range(4):
            c = p * 4 + b
            mb = b % 2
            if b < 2:
                pl.when(p > 0)(lambda mb=mb: wait_store(mb))
            else:
                wait_store(mb)
            def _pref(c=c, q=(b + 2) % 4):
                fire_idx(c + 2, q)

            if b < 2:
                _pref()
            else:
                pl.when(p < NP - 1)(_pref)
            def _gnext(c=c, q=(b + 1) % 4, nb=(b + 1) % 2):
                wait_idx(q)
                fire_gather(c + 1, q, nb)

            if b < 3:
                _gnext()
            else:
                pl.when(p < NP - 1)(_gnext)
            wait_gather(mb)
            g, o = rows[mb], outs[mb]

            @plsc.parallel_loop(0, SEQ, unroll=4)
            def _(r):
                for j in range(VPER):
                    sl = pl.ds(j * LANES, LANES)
                    o[r, sl] = g[r, sl] + pos_v[r, sl]

            fire_store(c, b % 2)
        return carry

    lax.fori_loop(0, NP, step, 0)
    wait_store(0)
    wait_store(1)


def kernel(x, emb_table):
    pos = jnp.asarray(_pos_encoding())
    table_pad = jnp.pad(emb_table, ((0, 0), (0, PAD - DIM)))
    x_flat = x.reshape(BATCH * SEQ).astype(jnp.int32)
    return _gather_kernel(x_flat, table_pad, pos)
